# baseline (speedup 1.0000x reference)
import jax
import jax.numpy as jnp
from jax.experimental import pallas as pl
from jax.experimental.pallas import tpu as pltpu

_BN = 1000


def _mm_body(m_ref, x_ref, o_ref):
    o_ref[...] = jax.lax.dot_general(
        m_ref[...],
        x_ref[...],
        dimension_numbers=(((1,), (1,)), ((), ())),
        preferred_element_type=jnp.float32,
    )


def kernel(inputs, targets, mem):
    del targets
    m, k = inputs.shape
    n = mem.shape[0]
    out_t = pl.pallas_call(
        _mm_body,
        grid=(n // _BN,),
        in_specs=[
            pl.BlockSpec((_BN, k), lambda i: (i, 0)),
            pl.BlockSpec((m, k), lambda i: (0, 0)),
        ],
        out_specs=pl.BlockSpec((_BN, m), lambda i: (i, 0)),
        out_shape=jax.ShapeDtypeStruct((n, m), jnp.float32),
        compiler_params=pltpu.CompilerParams(
            dimension_semantics=("arbitrary",),
        ),
    )(mem, inputs)
    return out_t.T


# speedup vs baseline: 1.2052x; 1.2052x over previous
import jax
import jax.numpy as jnp
from jax.experimental import pallas as pl
from jax.experimental.pallas import tpu as pltpu

_BN = 5000


def _mm_body(m_ref, x_ref, o_ref):
    o_ref[...] = jax.lax.dot_general(
        m_ref[...],
        x_ref[...],
        dimension_numbers=(((1,), (1,)), ((), ())),
        preferred_element_type=jnp.float32,
    )


def kernel(inputs, targets, mem):
    del targets
    m, k = inputs.shape
    n = mem.shape[0]
    out_t = pl.pallas_call(
        _mm_body,
        grid=(n // _BN,),
        in_specs=[
            pl.BlockSpec((_BN, k), lambda i: (i, 0)),
            pl.BlockSpec((m, k), lambda i: (0, 0)),
        ],
        out_specs=pl.BlockSpec((_BN, m), lambda i: (i, 0)),
        out_shape=jax.ShapeDtypeStruct((n, m), jnp.float32),
        compiler_params=pltpu.CompilerParams(
            dimension_semantics=("parallel",),
        ),
    )(mem, inputs)
    return out_t.T
